# baseline (speedup 1.0000x reference)
import functools

import jax
import jax.numpy as jnp
from jax.experimental import pallas as pl

N = 16384
C = 16
N_BINS = 50
EMB_DIM = 64
GROUP_SIZES = (6, 3, 3, 4)
GROUP_STARTS = (0, 6, 9, 12)
NQ = N_BINS - 1

BCH = 8192
OCH = 2048


def _extract_kernel(keys_ref, x_ref, j_ref, lw_ref, hw_ref,
                    e0_ref, e1_ref, e2_ref, e3_ref, out_ref):
    j2 = j_ref[:]
    top = jnp.uint32(0x80000000)

    def bit_step(i, p):
        shift = (jnp.int32(31) - i).astype(jnp.uint32)
        trial = p | (jnp.uint32(1) << shift)

        def chunk_step(c, cnt):
            kb = keys_ref[:, pl.ds(c * BCH, BCH)]
            lt = kb[None, :, :] < trial[:, :, None]
            return cnt + jnp.sum(lt.astype(jnp.int32), axis=-1)

        cnt = jax.lax.fori_loop(
            0, N // BCH, chunk_step, jnp.zeros((NQ, C), jnp.int32))
        return jnp.where(cnt <= j2, trial, p)

    p = jax.lax.fori_loop(0, 32, bit_step, jnp.zeros((NQ, C), jnp.uint32))

    imax = jnp.int32(0x7FFFFFFF)

    def succ_step(c, carry):
        cle, mgt = carry
        kb = keys_ref[:, pl.ds(c * BCH, BCH)]
        kb3 = kb[None, :, :]
        le = kb3 <= p[:, :, None]
        cle = cle + jnp.sum(le.astype(jnp.int32), axis=-1)
        kb3_i = jax.lax.bitcast_convert_type(kb3 ^ top, jnp.int32)
        gtv = jnp.where(le, imax, kb3_i)
        mgt = jnp.minimum(mgt, jnp.min(gtv, axis=-1))
        return cle, mgt

    cle, mgt = jax.lax.fori_loop(
        0, N // BCH, succ_step,
        (jnp.zeros((NQ, C), jnp.int32), jnp.full((NQ, C), imax)))
    mgt_u = jax.lax.bitcast_convert_type(mgt, jnp.uint32) ^ top
    p_hi = jnp.where(cle >= j2 + 2, p, mgt_u)

    def unmap(k):
        u = jnp.where(k >= top, k & jnp.uint32(0x7FFFFFFF), ~k)
        return jax.lax.bitcast_convert_type(u, jnp.float32)

    lo_v = unmap(p)
    hi_v = unmap(p_hi)

    edges = lo_v * lw_ref[:] + hi_v * hw_ref[:]

    embs = (e0_ref, e1_ref, e2_ref, e3_ref)

    def out_step(c, carry):
        xc = x_ref[:, pl.ds(c * OCH, OCH)]
        le = (edges[:, :, None] <= xc[None, :, :]
              ).astype(jnp.float32)
        for g in range(4):
            s = GROUP_STARTS[g]
            nf = GROUP_SIZES[g]
            sg = jnp.sum(le[:, s:s + nf, :], axis=1)
            a = jnp.concatenate(
                [jnp.float32(nf) - sg[:1], sg[:-1] - sg[1:], sg[-1:]],
                axis=0)
            oc = jnp.dot(embs[g][:], a,
                         preferred_element_type=jnp.float32)
            out_ref[g * EMB_DIM:(g + 1) * EMB_DIM,
                    pl.ds(c * OCH, OCH)] = oc * jnp.float32(1.0 / nf)
        return carry

    jax.lax.fori_loop(0, N // OCH, out_step, jnp.int32(0))


@functools.partial(jax.jit)
def kernel(features, emb_global, emb_hour, emb_session, emb_impression):
    xt = features.T
    u = jax.lax.bitcast_convert_type(xt, jnp.uint32)
    top = jnp.uint32(0x80000000)
    keys = jnp.where(u >= top, ~u, u | top)

    qs = jnp.linspace(0.0, 1.0, N_BINS + 1)[1:-1]
    q = qs * jnp.float32(N - 1)
    low = jnp.clip(jnp.floor(q), 0, N - 1)
    hw = (q - jnp.floor(q)).astype(jnp.float32)
    lw = (jnp.float32(1) - hw).astype(jnp.float32)
    jidx = low.astype(jnp.int32).reshape(NQ, 1)

    et = [e.T.astype(jnp.float32) for e in
          (emb_global, emb_hour, emb_session, emb_impression)]

    out_t = pl.pallas_call(
        _extract_kernel,
        out_shape=jax.ShapeDtypeStruct((4 * EMB_DIM, N), jnp.float32),
    )(keys, xt, jidx, lw.reshape(NQ, 1), hw.reshape(NQ, 1), *et)
    return out_t.T


# speedup vs baseline: 1.0024x; 1.0024x over previous
import functools

import jax
import jax.numpy as jnp
from jax.experimental import pallas as pl

N = 16384
C = 16
N_BINS = 50
EMB_DIM = 64
GROUP_SIZES = (6, 3, 3, 4)
GROUP_STARTS = (0, 6, 9, 12)
NQ = N_BINS - 1

BCH = 8192
OCH = 4096


def _extract_kernel(keys_ref, x_ref, j_ref, lw_ref, hw_ref,
                    e0_ref, e1_ref, e2_ref, e3_ref, out_ref):
    j2 = j_ref[:]
    top = jnp.uint32(0x80000000)

    def bit_step(i, p):
        shift = (jnp.int32(31) - i).astype(jnp.uint32)
        trial = p | (jnp.uint32(1) << shift)

        def chunk_step(c, cnt):
            kb = keys_ref[:, pl.ds(c * BCH, BCH)]
            lt = kb[None, :, :] < trial[:, :, None]
            return cnt + jnp.sum(lt.astype(jnp.int32), axis=-1)

        cnt = jax.lax.fori_loop(
            0, N // BCH, chunk_step, jnp.zeros((NQ, C), jnp.int32))
        return jnp.where(cnt <= j2, trial, p)

    p = jax.lax.fori_loop(0, 32, bit_step, jnp.zeros((NQ, C), jnp.uint32))

    imax = jnp.int32(0x7FFFFFFF)

    def succ_step(c, carry):
        cle, mgt = carry
        kb = keys_ref[:, pl.ds(c * BCH, BCH)]
        kb3 = kb[None, :, :]
        le = kb3 <= p[:, :, None]
        cle = cle + jnp.sum(le.astype(jnp.int32), axis=-1)
        kb3_i = jax.lax.bitcast_convert_type(kb3 ^ top, jnp.int32)
        gtv = jnp.where(le, imax, kb3_i)
        mgt = jnp.minimum(mgt, jnp.min(gtv, axis=-1))
        return cle, mgt

    cle, mgt = jax.lax.fori_loop(
        0, N // BCH, succ_step,
        (jnp.zeros((NQ, C), jnp.int32), jnp.full((NQ, C), imax)))
    mgt_u = jax.lax.bitcast_convert_type(mgt, jnp.uint32) ^ top
    p_hi = jnp.where(cle >= j2 + 2, p, mgt_u)

    def unmap(k):
        u = jnp.where(k >= top, k & jnp.uint32(0x7FFFFFFF), ~k)
        return jax.lax.bitcast_convert_type(u, jnp.float32)

    lo_v = unmap(p)
    hi_v = unmap(p_hi)

    edges = lo_v * lw_ref[:] + hi_v * hw_ref[:]

    embs = (e0_ref, e1_ref, e2_ref, e3_ref)

    def out_step(c, carry):
        xc = x_ref[:, pl.ds(c * OCH, OCH)]
        le = (edges[:, :, None] <= xc[None, :, :]
              ).astype(jnp.float32)
        for g in range(4):
            s = GROUP_STARTS[g]
            nf = GROUP_SIZES[g]
            sg = jnp.sum(le[:, s:s + nf, :], axis=1)
            a = jnp.concatenate(
                [jnp.float32(nf) - sg[:1], sg[:-1] - sg[1:], sg[-1:]],
                axis=0)
            oc = jnp.dot(embs[g][:], a,
                         preferred_element_type=jnp.float32)
            out_ref[g * EMB_DIM:(g + 1) * EMB_DIM,
                    pl.ds(c * OCH, OCH)] = oc * jnp.float32(1.0 / nf)
        return carry

    jax.lax.fori_loop(0, N // OCH, out_step, jnp.int32(0))


@functools.partial(jax.jit)
def kernel(features, emb_global, emb_hour, emb_session, emb_impression):
    xt = features.T
    u = jax.lax.bitcast_convert_type(xt, jnp.uint32)
    top = jnp.uint32(0x80000000)
    keys = jnp.where(u >= top, ~u, u | top)

    qs = jnp.linspace(0.0, 1.0, N_BINS + 1)[1:-1]
    q = qs * jnp.float32(N - 1)
    low = jnp.clip(jnp.floor(q), 0, N - 1)
    hw = (q - jnp.floor(q)).astype(jnp.float32)
    lw = (jnp.float32(1) - hw).astype(jnp.float32)
    jidx = low.astype(jnp.int32).reshape(NQ, 1)

    et = [e.T.astype(jnp.float32) for e in
          (emb_global, emb_hour, emb_session, emb_impression)]

    out_t = pl.pallas_call(
        _extract_kernel,
        out_shape=jax.ShapeDtypeStruct((4 * EMB_DIM, N), jnp.float32),
    )(keys, xt, jidx, lw.reshape(NQ, 1), hw.reshape(NQ, 1), *et)
    return out_t.T
